# baseline (speedup 1.0000x reference)
import functools

import jax
import jax.numpy as jnp
from jax import lax
from jax.experimental import pallas as pl
from jax.experimental.pallas import tpu as pltpu
from jax.experimental.pallas import tpu_sc as plsc

_N_INPUTS = 1008
_N_ATOMS = 16384
_N_CONFS = 2048
_BLK = 2048


_LOG2E = 1.4426950408889634
_LN2 = 0.6931471805599453


def _softplus(x):
    t = jnp.exp2(x * _LOG2E)
    return jnp.where(x > 20.0, x, jnp.log2(1.0 + t) * _LN2)

_NC = 2
_NS = 16
_NL = 16
_APT = _N_ATOMS // _NS
_CHUNKS = _APT // _NL
_ROWS = _N_CONFS // _NL


def _mlp_body(x_ref, w1_ref, b1_ref, w2_ref, b2_ref, w3_ref, b3_ref,
              w4_ref, b4_ref, out_ref):
    x = x_ref[...].astype(jnp.bfloat16)
    h = jnp.dot(x, w1_ref[...].astype(jnp.bfloat16),
                preferred_element_type=jnp.float32) + b1_ref[...]
    h = _softplus(h).astype(jnp.bfloat16)
    h = jnp.dot(h, w2_ref[...].astype(jnp.bfloat16),
                preferred_element_type=jnp.float32) + b2_ref[...]
    h = _softplus(h).astype(jnp.bfloat16)
    h = jnp.dot(h, w3_ref[...].astype(jnp.bfloat16),
                preferred_element_type=jnp.float32) + b3_ref[...]
    h = _softplus(h).astype(jnp.bfloat16)
    e = jnp.dot(h, w4_ref[...].astype(jnp.bfloat16),
                preferred_element_type=jnp.float32) + b4_ref[...]
    out_ref[...] = e


def _mlp_call(desc, w1t, b1, w2t, b2, w3t, b3, w4, b4):
    grid = _N_ATOMS // _BLK
    return pl.pallas_call(
        _mlp_body,
        grid=(grid,),
        in_specs=[
            pl.BlockSpec((_BLK, _N_INPUTS), lambda i: (i, 0)),
            pl.BlockSpec((_N_INPUTS, 512), lambda i: (0, 0)),
            pl.BlockSpec((1, 512), lambda i: (0, 0)),
            pl.BlockSpec((512, 256), lambda i: (0, 0)),
            pl.BlockSpec((1, 256), lambda i: (0, 0)),
            pl.BlockSpec((256, 128), lambda i: (0, 0)),
            pl.BlockSpec((1, 128), lambda i: (0, 0)),
            pl.BlockSpec((128, 1), lambda i: (0, 0)),
            pl.BlockSpec((1, 1), lambda i: (0, 0)),
        ],
        out_specs=pl.BlockSpec((_BLK, 1), lambda i: (i, 0)),
        out_shape=jax.ShapeDtypeStruct((_N_ATOMS, 1), jnp.float32),
    )(desc, w1t, b1, w2t, b2, w3t, b3, w4, b4)


_SLICE = _N_CONFS // _NS


def _sc_scatter_call(e1, e6, e7, e8, i1, i6, i7, i8, zeros):
    mesh = plsc.VectorSubcoreMesh(core_axis_name="c", subcore_axis_name="s")

    @functools.partial(
        pl.kernel,
        mesh=mesh,
        compiler_params=pltpu.CompilerParams(needs_layout_passes=False),
        out_type=jax.ShapeDtypeStruct((_NC, _N_CONFS), jnp.float32),
        scratch_types=[
            pltpu.VMEM((_APT,), jnp.int32),
            pltpu.VMEM((_APT,), jnp.float32),
            pltpu.VMEM((_N_CONFS,), jnp.float32),
            pltpu.VMEM((_SLICE,), jnp.float32),
            pltpu.VMEM((_SLICE,), jnp.float32),
            pltpu.VMEM_SHARED((_NS, _N_CONFS), jnp.float32),
        ],
    )
    def k(e1h, e6h, e7h, e8h, i1h, i6h, i7h, i8h, zh, outh,
          idx_v, val_v, acc, tmp_v, res_v, shared):
        cid = lax.axis_index("c")
        sid = lax.axis_index("s")
        pltpu.sync_copy(zh, acc)
        base = sid * _APT

        def process(eh, ih):
            pltpu.sync_copy(ih.at[pl.ds(base, _APT)], idx_v)
            pltpu.sync_copy(eh.at[pl.ds(base, _APT)], val_v)

            def chunk(c, carry):
                idx = idx_v[pl.ds(c * _NL, _NL)]
                val = val_v[pl.ds(c * _NL, _NL)]
                plsc.addupdate_scatter(acc, [idx], val)
                return carry

            lax.fori_loop(0, _CHUNKS, chunk, 0)

        @pl.when(cid == 0)
        def _():
            process(e1h, i1h)
            process(e6h, i6h)

        @pl.when(cid == 1)
        def _():
            process(e7h, i7h)
            process(e8h, i8h)

        pltpu.sync_copy(acc, shared.at[sid])
        plsc.subcore_barrier()

        zero16 = jnp.zeros((_NL,), jnp.float32)
        for j in range(_SLICE // _NL):
            res_v[pl.ds(j * _NL, _NL)] = zero16
        for t in range(_NS):
            pltpu.sync_copy(shared.at[t, pl.ds(sid * _SLICE, _SLICE)], tmp_v)
            for j in range(_SLICE // _NL):
                sl = pl.ds(j * _NL, _NL)
                res_v[sl] = res_v[sl] + tmp_v[sl]
        pltpu.sync_copy(res_v, outh.at[cid, pl.ds(sid * _SLICE, _SLICE)])

    return k(e1, e6, e7, e8, i1, i6, i7, i8, zeros)


def kernel(desc_1, desc_6, desc_7, desc_8, at2conf_1, at2conf_6,
           at2conf_7, at2conf_8, params, n_confs):
    es = []
    for t, desc in (("1", desc_1), ("6", desc_6), ("7", desc_7), ("8", desc_8)):
        (W1, b1), (W2, b2), (W3, b3), (W4, b4) = params[t]
        es.append(_mlp_call(
            desc,
            W1.T, b1.reshape(1, -1),
            W2.T, b2.reshape(1, -1),
            W3.T, b3.reshape(1, -1),
            W4.T, b4.reshape(1, 1),
        ).reshape(_N_ATOMS))
    zeros = jnp.zeros((_N_CONFS,), jnp.float32)
    part = _sc_scatter_call(es[0], es[1], es[2], es[3],
                            at2conf_1, at2conf_6, at2conf_7, at2conf_8,
                            zeros)
    return part.sum(axis=0)


# speedup vs baseline: 1.0061x; 1.0061x over previous
import functools

import jax
import jax.numpy as jnp
from jax import lax
from jax.experimental import pallas as pl
from jax.experimental.pallas import tpu as pltpu
from jax.experimental.pallas import tpu_sc as plsc

_N_INPUTS = 1008
_N_ATOMS = 16384
_N_CONFS = 2048
_BLK = 1024


_LOG2E = 1.4426950408889634
_LN2 = 0.6931471805599453


def _softplus(x):
    t = jnp.exp2(x * _LOG2E)
    return jnp.where(x > 20.0, x, jnp.log2(1.0 + t) * _LN2)

_NC = 2
_NS = 16
_NL = 16
_APT = _N_ATOMS // _NS
_CHUNKS = _APT // _NL
_ROWS = _N_CONFS // _NL


_NSTEPS = _N_ATOMS // _BLK


def _one_type(x_ref, w1_ref, b1_ref, w2_ref, b2_ref, w3_ref, b3_ref,
              w4_ref, b4_ref, out_ref):
    x = x_ref[...].astype(jnp.bfloat16)
    h = jnp.dot(x, w1_ref[...].astype(jnp.bfloat16),
                preferred_element_type=jnp.float32) + b1_ref[...]
    h = _softplus(h).astype(jnp.bfloat16)
    h = jnp.dot(h, w2_ref[...].astype(jnp.bfloat16),
                preferred_element_type=jnp.float32) + b2_ref[...]
    h = _softplus(h).astype(jnp.bfloat16)
    h = jnp.dot(h, w3_ref[...].astype(jnp.bfloat16),
                preferred_element_type=jnp.float32) + b3_ref[...]
    h = _softplus(h).astype(jnp.bfloat16)
    e = jnp.dot(h, w4_ref[...].astype(jnp.bfloat16),
                preferred_element_type=jnp.float32) + b4_ref[...]
    out_ref[0] = e


def _mlp_body(*refs):
    t = pl.program_id(0)
    descs = refs[0:4]
    out_ref = refs[-1]
    for k in range(4):
        ws = refs[4 + 8 * k: 4 + 8 * (k + 1)]

        @pl.when(t == k)
        def _(k=k, ws=ws):
            _one_type(descs[k], *ws, out_ref)


def _desc_map(k):
    def m(t, i):
        return (jnp.where(t < k, 0, jnp.where(t > k, _NSTEPS - 1, i)), 0)
    return m


def _mlp_call(descs, weights):
    d_specs = [pl.BlockSpec((_BLK, _N_INPUTS), _desc_map(k)) for k in range(4)]
    w_shapes = [(_N_INPUTS, 512), (1, 512), (512, 256), (1, 256),
                (256, 128), (1, 128), (128, 1), (1, 1)]
    w_specs = [pl.BlockSpec(s, lambda t, i: (0, 0)) for s in w_shapes] * 4
    return pl.pallas_call(
        _mlp_body,
        grid=(4, _NSTEPS),
        in_specs=d_specs + w_specs,
        out_specs=pl.BlockSpec((1, _BLK, 1), lambda t, i: (t, i, 0)),
        out_shape=jax.ShapeDtypeStruct((4, _N_ATOMS, 1), jnp.float32),
    )(*descs, *weights)


_SLICE = _N_CONFS // _NS


def _sc_scatter_call(e1, e6, e7, e8, i1, i6, i7, i8, zeros):
    mesh = plsc.VectorSubcoreMesh(core_axis_name="c", subcore_axis_name="s")

    @functools.partial(
        pl.kernel,
        mesh=mesh,
        compiler_params=pltpu.CompilerParams(needs_layout_passes=False),
        out_type=jax.ShapeDtypeStruct((_NC, _N_CONFS), jnp.float32),
        scratch_types=[
            pltpu.VMEM((_APT,), jnp.int32),
            pltpu.VMEM((_APT,), jnp.float32),
            pltpu.VMEM((_N_CONFS,), jnp.float32),
            pltpu.VMEM((_SLICE,), jnp.float32),
            pltpu.VMEM((_SLICE,), jnp.float32),
            pltpu.VMEM_SHARED((_NS, _N_CONFS), jnp.float32),
        ],
    )
    def k(e1h, e6h, e7h, e8h, i1h, i6h, i7h, i8h, zh, outh,
          idx_v, val_v, acc, tmp_v, res_v, shared):
        cid = lax.axis_index("c")
        sid = lax.axis_index("s")
        pltpu.sync_copy(zh, acc)
        base = sid * _APT

        def process(eh, ih):
            pltpu.sync_copy(ih.at[pl.ds(base, _APT)], idx_v)
            pltpu.sync_copy(eh.at[pl.ds(base, _APT)], val_v)

            def chunk(c, carry):
                idx = idx_v[pl.ds(c * _NL, _NL)]
                val = val_v[pl.ds(c * _NL, _NL)]
                plsc.addupdate_scatter(acc, [idx], val)
                return carry

            lax.fori_loop(0, _CHUNKS, chunk, 0)

        @pl.when(cid == 0)
        def _():
            process(e1h, i1h)
            process(e6h, i6h)

        @pl.when(cid == 1)
        def _():
            process(e7h, i7h)
            process(e8h, i8h)

        pltpu.sync_copy(acc, shared.at[sid])
        plsc.subcore_barrier()

        zero16 = jnp.zeros((_NL,), jnp.float32)
        for j in range(_SLICE // _NL):
            res_v[pl.ds(j * _NL, _NL)] = zero16
        for t in range(_NS):
            pltpu.sync_copy(shared.at[t, pl.ds(sid * _SLICE, _SLICE)], tmp_v)
            for j in range(_SLICE // _NL):
                sl = pl.ds(j * _NL, _NL)
                res_v[sl] = res_v[sl] + tmp_v[sl]
        pltpu.sync_copy(res_v, outh.at[cid, pl.ds(sid * _SLICE, _SLICE)])

    return k(e1, e6, e7, e8, i1, i6, i7, i8, zeros)


def kernel(desc_1, desc_6, desc_7, desc_8, at2conf_1, at2conf_6,
           at2conf_7, at2conf_8, params, n_confs):
    descs = [desc_1, desc_6, desc_7, desc_8]
    weights = []
    for t in ("1", "6", "7", "8"):
        (W1, b1), (W2, b2), (W3, b3), (W4, b4) = params[t]
        weights += [W1.T, b1.reshape(1, -1), W2.T, b2.reshape(1, -1),
                    W3.T, b3.reshape(1, -1), W4.T, b4.reshape(1, 1)]
    e_all = _mlp_call(descs, weights)
    es = [e_all[k].reshape(_N_ATOMS) for k in range(4)]
    zeros = jnp.zeros((_N_CONFS,), jnp.float32)
    part = _sc_scatter_call(es[0], es[1], es[2], es[3],
                            at2conf_1, at2conf_6, at2conf_7, at2conf_8,
                            zeros)
    return part.sum(axis=0)
